# baseline (speedup 1.0000x reference)
import functools
import jax
import jax.numpy as jnp
from jax import lax
from jax.experimental import pallas as pl
from jax.experimental.pallas import tpu as pltpu
from jax.experimental.pallas import tpu_sc as plsc

_VOCAB = 1000000
_EMB = 64
_BATCH = 4096
_SEQ = 200
_SEQP = 256

_NC = 2
_NS = 16
_L = 16
_NW = _NC * _NS
_RPW = _BATCH // _NW
_C0 = 96
_C1 = _SEQ - _C0


def _sc_body(x_hbm, table_hbm, out_hbm, idx_v, scale_v, rows_v, out_v,
             sems):
    wid = lax.axis_index("s") * _NC + lax.axis_index("c")
    base = wid * _RPW

    lane = lax.iota(jnp.int32, _L)
    zero = jnp.zeros((_L,), jnp.float32)
    rem = _SEQ - (_SEQ // _L) * _L
    rem_mask = jnp.minimum(jnp.maximum(lane - (_L - rem - 1), 0), 1)

    pltpu.sync_copy(x_hbm.at[pl.ds(base, _RPW)],
                    idx_v.at[:, pl.ds(0, _SEQ)])

    def transform_row(r, _):
        cnt = jnp.zeros((_L,), jnp.int32)
        vt = idx_v[r, pl.ds(_SEQ - _L, _L)]
        cnt = cnt + jnp.minimum(vt, 1) * rem_mask
        for k in range(_SEQ // _L):
            v = idx_v[r, pl.ds(k * _L, _L)]
            cnt = cnt + jnp.minimum(v, 1)
        cnt_s = jnp.sum(cnt)

        x = jnp.full((_L,), cnt_s.astype(jnp.float32) + jnp.float32(1e-10))
        i = plsc.bitcast(x, jnp.int32)
        i = jnp.int32(0x5F3759DF) - (i >> 1)
        y = plsc.bitcast(i, jnp.float32)
        half_x = x * jnp.float32(0.5)
        for _ in range(3):
            y = y * (jnp.float32(1.5) - half_x * y * y)
        scale_v[r, :] = x * y * jnp.float32(1.0 / _SEQ)
        return 0

    lax.fori_loop(0, _RPW, transform_row, 0)

    def issue_gathers(r, buf):
        pltpu.async_copy(
            table_hbm.at[idx_v.at[r, pl.ds(0, _C0)]],
            rows_v.at[buf, pl.ds(0, _C0)], sems.at[buf])
        pltpu.async_copy(
            table_hbm.at[idx_v.at[r, pl.ds(_C0, _C1)]],
            rows_v.at[buf, pl.ds(_C0, _C1)], sems.at[buf])

    def wait_gathers(r, buf):
        pltpu.make_async_copy(
            table_hbm.at[idx_v.at[r, pl.ds(0, _C0)]],
            rows_v.at[buf, pl.ds(0, _C0)], sems.at[buf]).wait()
        pltpu.make_async_copy(
            table_hbm.at[idx_v.at[r, pl.ds(_C0, _C1)]],
            rows_v.at[buf, pl.ds(_C0, _C1)], sems.at[buf]).wait()

    def process_row(r, buf):
        def acc_body(j, carry):
            a0, a1, a2, a3 = carry
            a0 = a0 + rows_v[buf, j, pl.ds(0, _L)]
            a1 = a1 + rows_v[buf, j, pl.ds(_L, _L)]
            a2 = a2 + rows_v[buf, j, pl.ds(2 * _L, _L)]
            a3 = a3 + rows_v[buf, j, pl.ds(3 * _L, _L)]
            return a0, a1, a2, a3

        a0, a1, a2, a3 = lax.fori_loop(
            0, _SEQ, acc_body, (zero, zero, zero, zero))

        scale = scale_v[r, :]
        out_v[r, pl.ds(0, _L)] = a0 * scale
        out_v[r, pl.ds(_L, _L)] = a1 * scale
        out_v[r, pl.ds(2 * _L, _L)] = a2 * scale
        out_v[r, pl.ds(3 * _L, _L)] = a3 * scale

    issue_gathers(0, 0)

    def row_loop(i, _):
        r = i * 2
        issue_gathers(r + 1, 1)
        wait_gathers(r, 0)
        process_row(r, 0)

        @pl.when(r + 2 < _RPW)
        def _():
            issue_gathers(r + 2, 0)

        wait_gathers(r + 1, 1)
        process_row(r + 1, 1)
        return 0

    lax.fori_loop(0, _RPW // 2, row_loop, 0)

    pltpu.sync_copy(out_v, out_hbm.at[pl.ds(base, _RPW)])


@jax.jit
def kernel(X, table):
    mesh = plsc.VectorSubcoreMesh(core_axis_name="c", subcore_axis_name="s")
    f = functools.partial(
        pl.kernel,
        out_type=jax.ShapeDtypeStruct((_BATCH, _EMB), jnp.float32),
        mesh=mesh,
        scratch_types=[
            pltpu.VMEM((_RPW, _SEQP), jnp.int32),
            pltpu.VMEM((_RPW, _L), jnp.float32),
            pltpu.VMEM((2, _SEQ, _EMB), jnp.float32),
            pltpu.VMEM((_RPW, _EMB), jnp.float32),
            pltpu.SemaphoreType.DMA((2,)),
        ],
        compiler_params=pltpu.CompilerParams(
            use_tc_tiling_on_sc=False, needs_layout_passes=False),
    )(_sc_body)
    return f(X, table)
